# Initial kernel scaffold; baseline (speedup 1.0000x reference)
import jax
import jax.numpy as jnp
from jax.experimental import pallas as pl


def kernel(x, embedding, Wg, W1, b1, W2, b2, Wp, bp):
    raise NotImplementedError("write your pallas kernel here")


# speedup vs baseline: 1.2245x; 1.2245x over previous
import functools

import jax
import jax.numpy as jnp
from jax.experimental import pallas as pl
from jax.experimental.pallas import tpu as pltpu

B, T, H, D = 2, 256, 8, 256
E, K, FFN = 8, 2, 1024
OUT = 5 * D
N = B * T * H
BLK = 256
EMB_N = T * H


def _moe_body(x_ref, emb_ref, wg_ref, w1_ref, b1_ref, w2_ref, b2_ref,
              wp_ref, bp_ref, out_ref):
    x32 = x_ref[...] + emb_ref[...]
    logits = jnp.dot(x32, wg_ref[...], preferred_element_type=jnp.float32)
    gates = jax.nn.softmax(logits, axis=-1)
    eidx = jax.lax.broadcasted_iota(jnp.int32, (BLK, E), 1)
    i1 = jnp.argmax(gates, axis=1)
    oh1 = (eidx == i1[:, None])
    v1 = jnp.max(gates, axis=1)
    g2 = jnp.where(oh1, -jnp.inf, gates)
    i2 = jnp.argmax(g2, axis=1)
    oh2 = (eidx == i2[:, None])
    v2 = jnp.max(g2, axis=1)
    s = v1 + v2
    mask = (oh1 * (v1 / s)[:, None] + oh2 * (v2 / s)[:, None]).astype(jnp.float32)

    xb = x32.astype(jnp.bfloat16)
    acc = jnp.zeros((BLK, D), jnp.float32)
    for e in range(E):
        h = jnp.dot(xb, w1_ref[e], preferred_element_type=jnp.float32)
        h = jax.nn.gelu(h + b1_ref[e][None, :])
        eo = jnp.dot(h.astype(jnp.bfloat16), w2_ref[e],
                     preferred_element_type=jnp.float32)
        eo = eo + b2_ref[e][None, :]
        acc = acc + mask[:, e][:, None] * eo
    y = jnp.dot(jax.nn.gelu(acc).astype(jnp.bfloat16), wp_ref[...],
                preferred_element_type=jnp.float32) + bp_ref[...]
    out_ref[...] = y


@jax.jit
def kernel(x, embedding, Wg, W1, b1, W2, b2, Wp, bp):
    xt = x.reshape(N, D)
    emb = embedding.reshape(EMB_N, D)
    grid = (N // BLK,)
    nb_e = EMB_N // BLK
    out = pl.pallas_call(
        _moe_body,
        grid=grid,
        in_specs=[
            pl.BlockSpec((BLK, D), lambda i: (i, 0)),
            pl.BlockSpec((BLK, D), lambda i: (jax.lax.rem(i, nb_e), 0)),
            pl.BlockSpec((D, E), lambda i: (0, 0)),
            pl.BlockSpec((E, D, FFN), lambda i: (0, 0, 0)),
            pl.BlockSpec((E, FFN), lambda i: (0, 0)),
            pl.BlockSpec((E, FFN, D), lambda i: (0, 0, 0)),
            pl.BlockSpec((E, D), lambda i: (0, 0)),
            pl.BlockSpec((D, OUT), lambda i: (0, 0)),
            pl.BlockSpec((1, OUT), lambda i: (0, 0)),
        ],
        out_specs=pl.BlockSpec((BLK, OUT), lambda i: (i, 0)),
        out_shape=jax.ShapeDtypeStruct((N, OUT), jnp.float32),
    )(xt, emb, Wg, W1.astype(jnp.bfloat16), b1, W2.astype(jnp.bfloat16),
      b2, Wp.astype(jnp.bfloat16), bp.reshape(1, OUT))
    return out.reshape(B, T, H, OUT)
